# baseline (speedup 1.0000x reference)
import functools

import jax
import jax.numpy as jnp
import numpy as np
from jax import lax
from jax.experimental import pallas as pl
from jax.experimental.pallas import tpu as pltpu
from jax.experimental.pallas import tpu_sc as plsc

NSEG = 8
FOLD = 24
L = 197
C = 768
NT = 512
NROWS = L * NT
NWORK = 32
RPW = NROWS // NWORK
NRING = 6
K = 24
GA = 3
PIECES = [K] * (RPW // K) + ([RPW % K] if RPW % K else [])


def _perm() -> np.ndarray:
    j = np.arange(NROWS, dtype=np.int64) // NT
    r = np.arange(NROWS, dtype=np.int64) % NT
    b, t = r // NSEG, r % NSEG
    ts = np.where((j >= 1) & (j < 1 + FOLD), (t - 1) % NSEG,
                  np.where(j >= L - FOLD, (t + 1) % NSEG, t))
    return (j * NT + b * NSEG + ts).astype(np.int32)


def _sc_body(x_hbm, perm_hbm, o_hbm, idx, buf, isem, gsem, ssem):
    wid = lax.axis_index("s") * 2 + lax.axis_index("c")
    base = wid * RPW

    c = pltpu.make_async_copy(perm_hbm.at[pl.ds(base, RPW)], idx, isem)
    c.start()
    c.wait()

    offs = []
    o = 0
    for n in PIECES:
        offs.append(o)
        o += n

    def gather(q):
        o, n = offs[q], PIECES[q]
        return pltpu.make_async_copy(
            x_hbm.at[idx.at[pl.ds(o, n)]],
            buf.at[q % NRING, 0:n, :],
            gsem.at[q % NRING],
        )

    def scatter(q):
        o, n = offs[q], PIECES[q]
        return pltpu.make_async_copy(
            buf.at[q % NRING, 0:n, :],
            o_hbm.at[pl.ds(base + o, n), :],
            ssem.at[q % NRING],
        )

    n = len(PIECES)
    for q in range(n):
        if q >= NRING:
            scatter(q - NRING).wait()
        gather(q).start()
        if q >= GA:
            gather(q - GA).wait()
            scatter(q - GA).start()
    for q in range(n - GA, n):
        gather(q).wait()
        scatter(q).start()
    for q in range(n - NRING, n):
        scatter(q).wait()


def kernel(x):
    nt, l, c = x.shape
    xt = jnp.transpose(x, (1, 0, 2)).reshape(l * nt, c)
    perm = jnp.asarray(_perm())
    run = functools.partial(
        pl.kernel,
        out_type=jax.ShapeDtypeStruct((l * nt, c), x.dtype),
        mesh=plsc.VectorSubcoreMesh(core_axis_name="c", subcore_axis_name="s"),
        scratch_types=[
            pltpu.VMEM((RPW,), jnp.int32),
            pltpu.VMEM((NRING, K, C), x.dtype),
            pltpu.SemaphoreType.DMA,
            pltpu.SemaphoreType.DMA((NRING,)),
            pltpu.SemaphoreType.DMA((NRING,)),
        ],
    )(_sc_body)
    out2 = run(xt, perm)
    return jnp.transpose(out2.reshape(l, nt, c), (1, 0, 2))


# speedup vs baseline: 1.0028x; 1.0028x over previous
import functools

import jax
import jax.numpy as jnp
import numpy as np
from jax import lax
from jax.experimental import pallas as pl
from jax.experimental.pallas import tpu as pltpu
from jax.experimental.pallas import tpu_sc as plsc

NSEG = 8
FOLD = 24
L = 197
C = 768
NT = 512
NROWS = L * NT
NWORK = 32
RPW = NROWS // NWORK
NRING = 6
K = 24
GA = 4
PIECES = [K] * (RPW // K) + ([RPW % K] if RPW % K else [])


def _perm() -> np.ndarray:
    j = np.arange(NROWS, dtype=np.int64) // NT
    r = np.arange(NROWS, dtype=np.int64) % NT
    b, t = r // NSEG, r % NSEG
    ts = np.where((j >= 1) & (j < 1 + FOLD), (t - 1) % NSEG,
                  np.where(j >= L - FOLD, (t + 1) % NSEG, t))
    return (j * NT + b * NSEG + ts).astype(np.int32)


def _sc_body(x_hbm, perm_hbm, o_hbm, idx, buf, isem, gsem, ssem):
    wid = lax.axis_index("s") * 2 + lax.axis_index("c")
    base = wid * RPW

    c = pltpu.make_async_copy(perm_hbm.at[pl.ds(base, RPW)], idx, isem)
    c.start()
    c.wait()

    offs = []
    o = 0
    for n in PIECES:
        offs.append(o)
        o += n

    def gather(q):
        o, n = offs[q], PIECES[q]
        return pltpu.make_async_copy(
            x_hbm.at[idx.at[pl.ds(o, n)]],
            buf.at[q % NRING, 0:n, :],
            gsem.at[q % NRING],
        )

    def scatter(q):
        o, n = offs[q], PIECES[q]
        return pltpu.make_async_copy(
            buf.at[q % NRING, 0:n, :],
            o_hbm.at[pl.ds(base + o, n), :],
            ssem.at[q % NRING],
        )

    n = len(PIECES)
    for q in range(n):
        if q >= NRING:
            scatter(q - NRING).wait()
        gather(q).start()
        if q >= GA:
            gather(q - GA).wait()
            scatter(q - GA).start()
    for q in range(n - GA, n):
        gather(q).wait()
        scatter(q).start()
    for q in range(n - NRING, n):
        scatter(q).wait()


def kernel(x):
    nt, l, c = x.shape
    xt = jnp.transpose(x, (1, 0, 2)).reshape(l * nt, c)
    perm = jnp.asarray(_perm())
    run = functools.partial(
        pl.kernel,
        out_type=jax.ShapeDtypeStruct((l * nt, c), x.dtype),
        mesh=plsc.VectorSubcoreMesh(core_axis_name="c", subcore_axis_name="s"),
        scratch_types=[
            pltpu.VMEM((RPW,), jnp.int32),
            pltpu.VMEM((NRING, K, C), x.dtype),
            pltpu.SemaphoreType.DMA,
            pltpu.SemaphoreType.DMA((NRING,)),
            pltpu.SemaphoreType.DMA((NRING,)),
        ],
    )(_sc_body)
    out2 = run(xt, perm)
    return jnp.transpose(out2.reshape(l, nt, c), (1, 0, 2))


# speedup vs baseline: 1.0039x; 1.0011x over previous
import functools

import jax
import jax.numpy as jnp
import numpy as np
from jax import lax
from jax.experimental import pallas as pl
from jax.experimental.pallas import tpu as pltpu
from jax.experimental.pallas import tpu_sc as plsc

NSEG = 8
FOLD = 24
L = 197
C = 768
NT = 512
NROWS = L * NT
NWORK = 32
RPW = NROWS // NWORK
NRING = 4
K = 32
GA = 2
PIECES = [K] * (RPW // K) + ([RPW % K] if RPW % K else [])


def _perm() -> np.ndarray:
    j = np.arange(NROWS, dtype=np.int64) // NT
    r = np.arange(NROWS, dtype=np.int64) % NT
    b, t = r // NSEG, r % NSEG
    ts = np.where((j >= 1) & (j < 1 + FOLD), (t - 1) % NSEG,
                  np.where(j >= L - FOLD, (t + 1) % NSEG, t))
    return (j * NT + b * NSEG + ts).astype(np.int32)


def _sc_body(x_hbm, perm_hbm, o_hbm, idx, buf, isem, gsem, ssem):
    wid = lax.axis_index("s") * 2 + lax.axis_index("c")
    base = wid * RPW

    c = pltpu.make_async_copy(perm_hbm.at[pl.ds(base, RPW)], idx, isem)
    c.start()
    c.wait()

    offs = []
    o = 0
    for n in PIECES:
        offs.append(o)
        o += n

    def gather(q):
        o, n = offs[q], PIECES[q]
        return pltpu.make_async_copy(
            x_hbm.at[idx.at[pl.ds(o, n)]],
            buf.at[q % NRING, 0:n, :],
            gsem.at[q % NRING],
        )

    def scatter(q):
        o, n = offs[q], PIECES[q]
        return pltpu.make_async_copy(
            buf.at[q % NRING, 0:n, :],
            o_hbm.at[pl.ds(base + o, n), :],
            ssem.at[q % NRING],
        )

    n = len(PIECES)
    for q in range(n):
        if q >= NRING:
            scatter(q - NRING).wait()
        gather(q).start()
        if q >= GA:
            gather(q - GA).wait()
            scatter(q - GA).start()
    for q in range(n - GA, n):
        gather(q).wait()
        scatter(q).start()
    for q in range(n - NRING, n):
        scatter(q).wait()


def kernel(x):
    nt, l, c = x.shape
    xt = jnp.transpose(x, (1, 0, 2)).reshape(l * nt, c)
    perm = jnp.asarray(_perm())
    run = functools.partial(
        pl.kernel,
        out_type=jax.ShapeDtypeStruct((l * nt, c), x.dtype),
        mesh=plsc.VectorSubcoreMesh(core_axis_name="c", subcore_axis_name="s"),
        scratch_types=[
            pltpu.VMEM((RPW,), jnp.int32),
            pltpu.VMEM((NRING, K, C), x.dtype),
            pltpu.SemaphoreType.DMA,
            pltpu.SemaphoreType.DMA((NRING,)),
            pltpu.SemaphoreType.DMA((NRING,)),
        ],
    )(_sc_body)
    out2 = run(xt, perm)
    return jnp.transpose(out2.reshape(l, nt, c), (1, 0, 2))
